# baseline (speedup 1.0000x reference)
import jax
import jax.numpy as jnp
from jax import lax
from jax.experimental import pallas as pl
from jax.experimental.pallas import tpu as pltpu
from jax.experimental.pallas import tpu_sc as plsc
from functools import partial

_NIN = 512
_NOUT = 8
_NDISC = 3
_OUT_STD = 0.1
_LO = -0.1
_HI = 0.1

_NW = 32
_RB = 16

_SC_ROWS = 16384


def _mu_tc_kernel(x_ref, wt_ref, o_ref):
    w = wt_ref[...]
    w0 = w[0]
    a = (w[1] - w0).astype(jnp.bfloat16)
    b = (w[2] - w[1]).astype(jnp.bfloat16)
    base = jnp.sum(w0, axis=0, keepdims=True)
    x = x_ref[...]
    m1 = (x > _LO).astype(jnp.bfloat16)
    m2 = (x > _HI).astype(jnp.bfloat16)
    dot = partial(jax.lax.dot_general,
                  dimension_numbers=(((1,), (0,)), ((), ())),
                  preferred_element_type=jnp.float32)
    o_ref[...] = dot(m1, a) + dot(m2, b) + base


def _mu_tc(x, W):
    batch = x.shape[0]
    tile = next(t for t in (2048, 1024, 512, 256, 128, 64, 32, 16, 8)
                if batch % t == 0)
    wt = jnp.transpose(W.reshape(_NOUT, _NIN, _NDISC), (2, 1, 0))
    return pl.pallas_call(
        _mu_tc_kernel,
        grid=(batch // tile,),
        in_specs=[
            pl.BlockSpec((tile, _NIN), lambda i: (i, 0)),
            pl.BlockSpec((_NDISC, _NIN, _NOUT), lambda i: (0, 0, 0)),
        ],
        out_specs=pl.BlockSpec((tile, _NOUT), lambda i: (i, 0)),
        out_shape=jax.ShapeDtypeStruct((batch, _NOUT), jnp.float32),
    )(x, wt)


def _mu_sc(x, W):
    batch = x.shape[0]
    rows_per_w = batch // _NW
    nblk = rows_per_w // _RB
    mesh = plsc.VectorSubcoreMesh(core_axis_name="c", subcore_axis_name="s")

    _PITCH = _NIN + 1

    @partial(
        pl.kernel,
        mesh=mesh,
        out_type=jax.ShapeDtypeStruct((batch * _NOUT,), jnp.float32),
        scratch_types=[
            pltpu.VMEM((2 * _RB, _PITCH), jnp.float32),
            pltpu.VMEM((_NOUT * _NIN * _NDISC,), jnp.float32),
            pltpu.VMEM((_RB * _NOUT,), jnp.float32),
            pltpu.SemaphoreType.DMA,
            pltpu.SemaphoreType.DMA,
        ],
        compiler_params=pltpu.CompilerParams(needs_layout_passes=False),
    )
    def sc_k(x_hbm, w_hbm, out_hbm, xbuf, wbuf, obuf, sem0, sem1):
        wid = lax.axis_index("s") * 2 + lax.axis_index("c")
        base_row = wid * rows_per_w
        pltpu.sync_copy(w_hbm, wbuf)
        lanes = lax.iota(jnp.int32, 16)
        out_off = lanes * _NOUT
        sems = (sem0, sem1)

        def x_slice(blk):
            return x_hbm.at[pl.ds(base_row + blk * _RB, _RB), :]

        def half(p):
            return xbuf.at[pl.ds(p * _RB, _RB), pl.ds(0, _NIN)]

        pltpu.async_copy(x_slice(0), half(0), sem0)

        def do_pair(t, carry):
            for p in range(2):
                blk = 2 * t + p
                pltpu.make_async_copy(x_slice(0), half(p), sems[p]).wait()
                nxt = jnp.minimum(blk + 1, nblk - 1)
                pltpu.async_copy(x_slice(nxt), half(1 - p), sems[1 - p])
                row_idx = lanes + p * _RB

                def col_body(i, accs):
                    xv = plsc.load_gather(xbuf, [row_idx, jnp.full((16,), i, jnp.int32)])
                    d = ((xv > _LO).astype(jnp.int32)
                         + (xv > _HI).astype(jnp.int32))
                    idx = d + 3 * i
                    return tuple(
                        accs[o] + plsc.load_gather(
                            wbuf, [idx + o * (_NIN * _NDISC)])
                        for o in range(_NOUT))

                zeros = tuple(jnp.zeros((16,), jnp.float32)
                              for _ in range(_NOUT))
                accs = plsc.parallel_loop(
                    0, _NIN, unroll=8, carry=zeros)(col_body)
                for o in range(_NOUT):
                    plsc.store_scatter(obuf, [out_off + o], accs[o])
                pltpu.sync_copy(
                    obuf,
                    out_hbm.at[pl.ds((base_row + blk * _RB) * _NOUT,
                                     _RB * _NOUT)])
            return carry

        lax.fori_loop(0, nblk // 2, do_pair, 0)
        pltpu.make_async_copy(x_slice(0), half(0), sem0).wait()

    out = sc_k(x, W.reshape(-1))
    return out.reshape(batch, _NOUT)


def kernel(x, W):
    batch = x.shape[0]
    s = min(_SC_ROWS, batch)
    if s == batch:
        mu = _mu_sc(x, W)
    elif s == 0:
        mu = _mu_tc(x, W)
    else:
        mu_sc = _mu_sc(x[:s], W)
        mu_tc = _mu_tc(x[s:], W)
        mu = jnp.concatenate([mu_sc, mu_tc], axis=0)
    idx = jnp.arange(_NOUT)
    scale_tril = (jnp.zeros((1, _NOUT, _NOUT), dtype=jnp.float32)
                  .at[:, idx, idx].set(_OUT_STD))
    return mu, scale_tril


# speedup vs baseline: 3.3620x; 3.3620x over previous
import jax
import jax.numpy as jnp
from jax import lax
from jax.experimental import pallas as pl
from jax.experimental.pallas import tpu as pltpu
from jax.experimental.pallas import tpu_sc as plsc
from functools import partial

_NIN = 512
_NOUT = 8
_NDISC = 3
_OUT_STD = 0.1
_LO = -0.1
_HI = 0.1

_NW = 32
_RB = 16

_SC_ROWS = 1024


def _mu_tc_kernel(x_ref, wt_ref, o_ref):
    w = wt_ref[...]
    w0 = w[0]
    a = (w[1] - w0).astype(jnp.bfloat16)
    b = (w[2] - w[1]).astype(jnp.bfloat16)
    base = jnp.sum(w0, axis=0, keepdims=True)
    x = x_ref[...]
    m1 = (x > _LO).astype(jnp.bfloat16)
    m2 = (x > _HI).astype(jnp.bfloat16)
    dot = partial(jax.lax.dot_general,
                  dimension_numbers=(((1,), (0,)), ((), ())),
                  preferred_element_type=jnp.float32)
    o_ref[...] = dot(m1, a) + dot(m2, b) + base


def _mu_tc(x, W):
    batch = x.shape[0]
    tile = next(t for t in (2048, 1024, 512, 256, 128, 64, 32, 16, 8)
                if batch % t == 0)
    wt = jnp.transpose(W.reshape(_NOUT, _NIN, _NDISC), (2, 1, 0))
    return pl.pallas_call(
        _mu_tc_kernel,
        grid=(batch // tile,),
        in_specs=[
            pl.BlockSpec((tile, _NIN), lambda i: (i, 0)),
            pl.BlockSpec((_NDISC, _NIN, _NOUT), lambda i: (0, 0, 0)),
        ],
        out_specs=pl.BlockSpec((tile, _NOUT), lambda i: (i, 0)),
        out_shape=jax.ShapeDtypeStruct((batch, _NOUT), jnp.float32),
    )(x, wt)


def _mu_sc(x, W):
    batch = x.shape[0]
    rows_per_w = batch // _NW
    nblk = rows_per_w // _RB
    mesh = plsc.VectorSubcoreMesh(core_axis_name="c", subcore_axis_name="s")

    _PITCH = _NIN + 1

    @partial(
        pl.kernel,
        mesh=mesh,
        out_type=jax.ShapeDtypeStruct((batch * _NOUT,), jnp.float32),
        scratch_types=[
            pltpu.VMEM((2 * _RB, _PITCH), jnp.float32),
            pltpu.VMEM((_NOUT * _NIN * _NDISC,), jnp.float32),
            pltpu.VMEM((_RB * _NOUT,), jnp.float32),
            pltpu.SemaphoreType.DMA,
            pltpu.SemaphoreType.DMA,
        ],
        compiler_params=pltpu.CompilerParams(needs_layout_passes=False),
    )
    def sc_k(x_hbm, w_hbm, out_hbm, xbuf, wbuf, obuf, sem0, sem1):
        wid = lax.axis_index("s") * 2 + lax.axis_index("c")
        base_row = wid * rows_per_w
        pltpu.sync_copy(w_hbm, wbuf)
        lanes = lax.iota(jnp.int32, 16)
        out_off = lanes * _NOUT
        sems = (sem0, sem1)

        def x_slice(blk):
            return x_hbm.at[pl.ds(base_row + blk * _RB, _RB), :]

        def half(p):
            return xbuf.at[pl.ds(p * _RB, _RB), pl.ds(0, _NIN)]

        pltpu.async_copy(x_slice(0), half(0), sem0)

        def do_pair(t, carry):
            for p in range(2):
                blk = 2 * t + p
                pltpu.make_async_copy(x_slice(0), half(p), sems[p]).wait()
                nxt = jnp.minimum(blk + 1, nblk - 1)
                pltpu.async_copy(x_slice(nxt), half(1 - p), sems[1 - p])
                row_idx = lanes + p * _RB

                def col_body(i, accs):
                    xv = plsc.load_gather(xbuf, [row_idx, jnp.full((16,), i, jnp.int32)])
                    d = ((xv > _LO).astype(jnp.int32)
                         + (xv > _HI).astype(jnp.int32))
                    idx = d + 3 * i
                    return tuple(
                        accs[o] + plsc.load_gather(
                            wbuf, [idx + o * (_NIN * _NDISC)])
                        for o in range(_NOUT))

                zeros = tuple(jnp.zeros((16,), jnp.float32)
                              for _ in range(_NOUT))
                accs = plsc.parallel_loop(
                    0, _NIN, unroll=8, carry=zeros)(col_body)
                for o in range(_NOUT):
                    plsc.store_scatter(obuf, [out_off + o], accs[o])
                pltpu.sync_copy(
                    obuf,
                    out_hbm.at[pl.ds((base_row + blk * _RB) * _NOUT,
                                     _RB * _NOUT)])
            return carry

        lax.fori_loop(0, nblk // 2, do_pair, 0)
        pltpu.make_async_copy(x_slice(0), half(0), sem0).wait()

    out = sc_k(x, W.reshape(-1))
    return out.reshape(batch, _NOUT)


def kernel(x, W):
    batch = x.shape[0]
    s = min(_SC_ROWS, batch)
    if s == batch:
        mu = _mu_sc(x, W)
    elif s == 0:
        mu = _mu_tc(x, W)
    else:
        mu_sc = _mu_sc(x[:s], W)
        mu_tc = _mu_tc(x[s:], W)
        mu = jnp.concatenate([mu_sc, mu_tc], axis=0)
    idx = jnp.arange(_NOUT)
    scale_tril = (jnp.zeros((1, _NOUT, _NOUT), dtype=jnp.float32)
                  .at[:, idx, idx].set(_OUT_STD))
    return mu, scale_tril


# speedup vs baseline: 5.0339x; 1.4973x over previous
import jax
import jax.numpy as jnp
from jax import lax
from jax.experimental import pallas as pl
from jax.experimental.pallas import tpu as pltpu
from jax.experimental.pallas import tpu_sc as plsc
from functools import partial

_NIN = 512
_NOUT = 8
_NDISC = 3
_OUT_STD = 0.1
_LO = -0.1
_HI = 0.1

_NW = 32
_RB = 16

_SC_ROWS = 1024


def _mu_tc_kernel(x_ref, wt_ref, o_ref):
    w = wt_ref[...]
    w0 = w[0]
    a = (w[1] - w0).astype(jnp.bfloat16)
    b = (w[2] - w[1]).astype(jnp.bfloat16)
    base = jnp.sum(w0, axis=0, keepdims=True)
    x = x_ref[...]
    m1 = (x > _LO).astype(jnp.bfloat16)
    m2 = (x > _HI).astype(jnp.bfloat16)
    dot = partial(jax.lax.dot_general,
                  dimension_numbers=(((1,), (0,)), ((), ())),
                  preferred_element_type=jnp.float32)
    o_ref[...] = dot(m1, a) + dot(m2, b) + base


def _mu_tc(x, W, row_start=0):
    batch = x.shape[0]
    nrows = batch - row_start
    tile = next(t for t in (2048, 1024, 512, 256, 128, 64, 32, 16, 8)
                if nrows % t == 0 and row_start % t == 0)
    blk0 = row_start // tile
    wt = jnp.transpose(W.reshape(_NOUT, _NIN, _NDISC), (2, 1, 0))
    return pl.pallas_call(
        _mu_tc_kernel,
        grid=(nrows // tile,),
        in_specs=[
            pl.BlockSpec((tile, _NIN), lambda i: (i + blk0, 0)),
            pl.BlockSpec((_NDISC, _NIN, _NOUT), lambda i: (0, 0, 0)),
        ],
        out_specs=pl.BlockSpec((tile, _NOUT), lambda i: (i, 0)),
        out_shape=jax.ShapeDtypeStruct((nrows, _NOUT), jnp.float32),
    )(x, wt)


def _mu_sc(x, W, nrows):
    rows_per_w = nrows // _NW
    nblk = rows_per_w // _RB
    mesh = plsc.VectorSubcoreMesh(core_axis_name="c", subcore_axis_name="s")

    _PITCH = _NIN + 1

    @partial(
        pl.kernel,
        mesh=mesh,
        out_type=jax.ShapeDtypeStruct((nrows * _NOUT,), jnp.float32),
        scratch_types=[
            pltpu.VMEM((2 * _RB, _PITCH), jnp.float32),
            pltpu.VMEM((_NOUT * _NIN * _NDISC,), jnp.float32),
            pltpu.VMEM((_RB * _NOUT,), jnp.float32),
            pltpu.SemaphoreType.DMA,
            pltpu.SemaphoreType.DMA,
        ],
        compiler_params=pltpu.CompilerParams(needs_layout_passes=False),
    )
    def sc_k(x_hbm, w_hbm, out_hbm, xbuf, wbuf, obuf, sem0, sem1):
        wid = lax.axis_index("s") * 2 + lax.axis_index("c")
        base_row = wid * rows_per_w
        pltpu.sync_copy(w_hbm, wbuf)
        lanes = lax.iota(jnp.int32, 16)
        out_off = lanes * _NOUT
        sems = (sem0, sem1)

        def x_slice(blk):
            return x_hbm.at[pl.ds(base_row + blk * _RB, _RB), :]

        def half(p):
            return xbuf.at[pl.ds(p * _RB, _RB), pl.ds(0, _NIN)]

        pltpu.async_copy(x_slice(0), half(0), sem0)

        def do_pair(t, carry):
            for p in range(2):
                blk = 2 * t + p
                pltpu.make_async_copy(x_slice(0), half(p), sems[p]).wait()
                nxt = jnp.minimum(blk + 1, nblk - 1)
                pltpu.async_copy(x_slice(nxt), half(1 - p), sems[1 - p])
                row_idx = lanes + p * _RB

                def col_body(i, accs):
                    xv = plsc.load_gather(xbuf, [row_idx, jnp.full((16,), i, jnp.int32)])
                    d = ((xv > _LO).astype(jnp.int32)
                         + (xv > _HI).astype(jnp.int32))
                    idx = d + 3 * i
                    return tuple(
                        accs[o] + plsc.load_gather(
                            wbuf, [idx + o * (_NIN * _NDISC)])
                        for o in range(_NOUT))

                zeros = tuple(jnp.zeros((16,), jnp.float32)
                              for _ in range(_NOUT))
                accs = plsc.parallel_loop(
                    0, _NIN, unroll=8, carry=zeros)(col_body)
                for o in range(_NOUT):
                    plsc.store_scatter(obuf, [out_off + o], accs[o])
                pltpu.sync_copy(
                    obuf,
                    out_hbm.at[pl.ds((base_row + blk * _RB) * _NOUT,
                                     _RB * _NOUT)])
            return carry

        lax.fori_loop(0, nblk // 2, do_pair, 0)
        pltpu.make_async_copy(x_slice(0), half(0), sem0).wait()

    out = sc_k(x, W.reshape(-1))
    return out.reshape(nrows, _NOUT)


def kernel(x, W):
    batch = x.shape[0]
    s = min(_SC_ROWS, batch)
    if s == batch:
        mu = _mu_sc(x, W, batch)
    elif s == 0:
        mu = _mu_tc(x, W)
    else:
        mu_sc = _mu_sc(x, W, s)
        mu_tc = _mu_tc(x, W, s)
        mu = jnp.concatenate([mu_sc, mu_tc], axis=0)
    idx = jnp.arange(_NOUT)
    scale_tril = (jnp.zeros((1, _NOUT, _NOUT), dtype=jnp.float32)
                  .at[:, idx, idx].set(_OUT_STD))
    return mu, scale_tril
